# baseline (speedup 1.0000x reference)
import functools

import jax
import jax.numpy as jnp
from jax import lax
from jax.experimental import pallas as pl
from jax.experimental.pallas import tpu as pltpu
from jax.experimental.pallas import tpu_sc as plsc

NUM_EMB = 1000000
DIM = 32
BATCH = 16384

NC = 2
NS = 16
NW = NC * NS
BPW = BATCH // NW
IDX_MINOR = 128
NJ = BPW // IDX_MINOR


def _sc_gather_body(x_hbm, table_hbm, out_hbm, idx_v, rows_v, sem):
    wid = lax.axis_index("s") * NC + lax.axis_index("c")
    base = wid * BPW

    pltpu.sync_copy(x_hbm.at[wid], idx_v)

    copies = [
        pltpu.async_copy(
            table_hbm.at[idx_v.at[j]],
            rows_v.at[pl.ds(j * IDX_MINOR, IDX_MINOR)],
            sem,
        )
        for j in range(NJ)
    ]
    for c in copies:
        c.wait()

    pltpu.sync_copy(rows_v, out_hbm.at[pl.ds(base, BPW)])


def _tc_dense_body(rows_ref, wt_ref, b_ref, out_ref):
    rows = rows_ref[...]
    wt = wt_ref[...]
    acc = jnp.sum(rows * wt, axis=1, keepdims=True) + b_ref[0, 0]
    out_ref[...] = 1.0 / (1.0 + jnp.exp(-acc))


TC_BLK = 2048
TC_GRID = BATCH // TC_BLK


@jax.jit
def _run(x3, table, wt, b2):
    mesh = plsc.VectorSubcoreMesh(core_axis_name="c", subcore_axis_name="s")
    gather = functools.partial(
        pl.kernel,
        mesh=mesh,
        compiler_params=pltpu.CompilerParams(use_tc_tiling_on_sc=False),
        out_type=jax.ShapeDtypeStruct((BATCH, DIM), jnp.float32),
        scratch_types=[
            pltpu.VMEM((NJ, IDX_MINOR), jnp.int32),
            pltpu.VMEM((BPW, DIM), jnp.float32),
            pltpu.SemaphoreType.DMA,
        ],
    )(_sc_gather_body)
    rows = gather(x3, table)

    dense = pl.pallas_call(
        _tc_dense_body,
        grid=(TC_GRID,),
        in_specs=[
            pl.BlockSpec((TC_BLK, DIM), lambda i: (i, 0)),
            pl.BlockSpec((1, DIM), lambda i: (0, 0)),
            pl.BlockSpec(memory_space=pltpu.SMEM),
        ],
        out_specs=pl.BlockSpec((TC_BLK, 1), lambda i: (i, 0)),
        out_shape=jax.ShapeDtypeStruct((BATCH, 1), jnp.float32),
    )
    return dense(rows, wt, b2)


def kernel(x, table, W, b):
    x3 = x.astype(jnp.int32).reshape(NW, NJ, IDX_MINOR)
    wt = W.reshape(1, DIM)
    b2 = b.reshape(1, 1)
    return _run(x3, table, wt, b2)
